# Initial kernel scaffold; baseline (speedup 1.0000x reference)
import functools

import jax
import jax.numpy as jnp
from jax import lax
from jax.experimental import pallas as pl
from jax.experimental.pallas import tpu as pltpu
from jax.experimental.pallas import tpu_sc as plsc

N = 10000
E = 320000
D = 128

NC = 2
NS = 16
NW = NC * NS
K = 128
NCHUNK = E // K
CHUNKS_PER_W = -(-NCHUNK // NW)
HB_ROWS = 80
RPW = N // NS

_mesh = plsc.VectorSubcoreMesh(
    core_axis_name="c", subcore_axis_name="s", num_cores=NC, num_subcores=NS
)


@functools.partial(
    pl.kernel,
    out_type=jax.ShapeDtypeStruct((NC, HB_ROWS, D), jnp.float32),
    mesh=_mesh,
    scratch_types=[
        pltpu.VMEM((HB_ROWS, D), jnp.float32),
        pltpu.VMEM((K,), jnp.int32),
        pltpu.VMEM((HB_ROWS,), jnp.int32),
        pltpu.VMEM_SHARED((HB_ROWS, D), jnp.float32),
    ],
)
def _deg_call(dst_hbm, out_hbm, hist_v, idx_v, rowid_v, hist_sh):
    cid = lax.axis_index("c")
    sid = lax.axis_index("s")
    wid = sid * NC + cid

    zero16 = jnp.zeros((16,), jnp.float32)
    iota16 = lax.iota(jnp.int32, 16)
    ones16 = jnp.ones((16,), jnp.float32)

    def _zb(i, carry):
        hist_v[i >> 3, pl.ds(pl.multiple_of((i & 7) * 16, 16), 16)] = zero16
        return carry

    lax.fori_loop(0, HB_ROWS * (D // 16), _zb, 0)

    def _ib(i, carry):
        rowid_v[pl.ds(pl.multiple_of(i * 16, 16), 16)] = iota16 + i * 16
        return carry

    lax.fori_loop(0, HB_ROWS // 16, _ib, 0)

    @pl.when(sid == 0)
    def _():
        pltpu.sync_copy(hist_v, hist_sh)

    plsc.subcore_barrier()

    def _chunk(t, carry):
        chunk = t * NW + wid

        @pl.when(chunk < NCHUNK)
        def _():
            base = pl.multiple_of(chunk * K, K)
            pltpu.sync_copy(dst_hbm.at[pl.ds(base, K)], idx_v)

            def _sub(j, c2):
                idx16 = idx_v[pl.ds(pl.multiple_of(j * 16, 16), 16)]
                plsc.addupdate_scatter(
                    hist_v, [idx16 >> 7, idx16 & 127], ones16
                )
                return c2

            lax.fori_loop(0, K // 16, _sub, 0)

        return carry

    lax.fori_loop(0, CHUNKS_PER_W, _chunk, 0)

    pltpu.sync_copy(hist_v, hist_sh.at[rowid_v], add=True)
    plsc.subcore_barrier()
    rows_per_tile = HB_ROWS // NS
    pltpu.sync_copy(
        hist_sh.at[pl.ds(sid * rows_per_tile, rows_per_tile)],
        out_hbm.at[cid, pl.ds(sid * rows_per_tile, rows_per_tile)],
    )


@functools.partial(
    pl.kernel,
    out_type=jax.ShapeDtypeStruct((NC, N, D), jnp.float32),
    mesh=_mesh,
    scratch_types=[
        pltpu.VMEM((K,), jnp.int32),
        pltpu.VMEM((K,), jnp.int32),
        pltpu.VMEM((K, D), jnp.float32),
        pltpu.VMEM_SHARED((N, D), jnp.float32),
        pltpu.SemaphoreType.DMA,
    ],
)
def _scatter_call(g_hbm, src_hbm, dst_hbm, out_hbm, srcv, dstv, rows, acc, sem):
    cid = lax.axis_index("c")
    sid = lax.axis_index("s")
    wid = sid * NC + cid

    zero16 = jnp.zeros((16,), jnp.float32)

    def _zb(i, carry):
        rows[i >> 3, pl.ds(pl.multiple_of((i & 7) * 16, 16), 16)] = zero16
        return carry

    lax.fori_loop(0, K * (D // 16), _zb, 0)

    def _za(k, carry):
        pltpu.sync_copy(
            rows.at[pl.ds(0, RPW // 5)],
            acc.at[pl.ds(sid * RPW + k * (RPW // 5), RPW // 5)],
        )
        return carry

    lax.fori_loop(0, 5, _za, 0)
    plsc.subcore_barrier()

    def _chunk(t, carry):
        chunk = t * NW + wid

        @pl.when(chunk < NCHUNK)
        def _():
            base = pl.multiple_of(chunk * K, K)
            pltpu.sync_copy(src_hbm.at[pl.ds(base, K)], srcv)
            pltpu.sync_copy(dst_hbm.at[pl.ds(base, K)], dstv)
            pltpu.async_copy(g_hbm.at[srcv], rows, sem).wait()
            pltpu.sync_copy(rows, acc.at[dstv], add=True)

        return carry

    lax.fori_loop(0, CHUNKS_PER_W, _chunk, 0)
    plsc.subcore_barrier()

    pltpu.sync_copy(
        acc.at[pl.ds(sid * RPW, RPW)], out_hbm.at[cid, pl.ds(sid * RPW, RPW)]
    )


_ROWS_BLK = 2000


def _mm_body(x_ref, w_ref, deg_ref, g_ref):
    dinv = lax.rsqrt(deg_ref[...])
    g_ref[...] = (
        jnp.dot(x_ref[...], w_ref[...], preferred_element_type=jnp.float32)
        * dinv
    )


def _fin_body(s0_ref, s1_ref, g_ref, deg_ref, b_ref, o_ref):
    dinv = lax.rsqrt(deg_ref[...])
    z = (s0_ref[...] + s1_ref[...] + g_ref[...]) * dinv + b_ref[...]
    o_ref[...] = jnp.where(z > 0, z, jnp.exp(jnp.minimum(z, 0.0)) - 1.0)


def kernel(x, edge_index, W, b):
    src = edge_index[0]
    dst = edge_index[1]

    hp = _deg_call(dst)
    deg = (hp[0] + hp[1]).reshape(-1)[:N] + 1.0
    deg_b = jnp.broadcast_to(deg[:, None], (N, D))

    g = pl.pallas_call(
        _mm_body,
        grid=(N // _ROWS_BLK,),
        in_specs=[
            pl.BlockSpec((_ROWS_BLK, D), lambda i: (i, 0)),
            pl.BlockSpec((D, D), lambda i: (0, 0)),
            pl.BlockSpec((_ROWS_BLK, D), lambda i: (i, 0)),
        ],
        out_specs=pl.BlockSpec((_ROWS_BLK, D), lambda i: (i, 0)),
        out_shape=jax.ShapeDtypeStruct((N, D), jnp.float32),
    )(x, W, deg_b)

    s = _scatter_call(g, src, dst)

    out = pl.pallas_call(
        _fin_body,
        grid=(N // _ROWS_BLK,),
        in_specs=[
            pl.BlockSpec((_ROWS_BLK, D), lambda i: (i, 0)),
            pl.BlockSpec((_ROWS_BLK, D), lambda i: (i, 0)),
            pl.BlockSpec((_ROWS_BLK, D), lambda i: (i, 0)),
            pl.BlockSpec((_ROWS_BLK, D), lambda i: (i, 0)),
            pl.BlockSpec((1, D), lambda i: (0, 0)),
        ],
        out_specs=pl.BlockSpec((_ROWS_BLK, D), lambda i: (i, 0)),
        out_shape=jax.ShapeDtypeStruct((N, D), jnp.float32),
    )(s[0], s[1], g, deg_b, b.reshape(1, D))

    return out


# speedup vs baseline: 21.6744x; 21.6744x over previous
import functools

import jax
import jax.numpy as jnp
from jax import lax
from jax.experimental import pallas as pl
from jax.experimental.pallas import tpu as pltpu
from jax.experimental.pallas import tpu_sc as plsc

N = 10000
E = 320000
D = 128

NC = 2
NS = 16
NW = NC * NS
K = 128
NCHUNK = E // K
CHUNKS_PER_W = -(-NCHUNK // NW)
HB_ROWS = 128
NPAD = 10240
RPW = NPAD // NS

_mesh = plsc.VectorSubcoreMesh(
    core_axis_name="c", subcore_axis_name="s", num_cores=NC, num_subcores=NS
)


NBINS = HB_ROWS * D
BPT = NBINS // NS


@functools.partial(
    pl.kernel,
    out_type=jax.ShapeDtypeStruct((NC, NBINS), jnp.float32),
    mesh=_mesh,
    scratch_types=[
        pltpu.VMEM((NBINS,), jnp.int32),
        pltpu.VMEM((K,), jnp.int32),
        pltpu.VMEM((NS, BPT), jnp.int32),
        pltpu.VMEM((BPT,), jnp.float32),
        pltpu.VMEM_SHARED((NS, NBINS), jnp.int32),
    ],
    compiler_params=pltpu.CompilerParams(needs_layout_passes=False),
)
def _deg_call(dst_hbm, out_hbm, hist_v, idx_v, red_v, res_v, stage_sh):
    cid = lax.axis_index("c")
    sid = lax.axis_index("s")
    wid = sid * NC + cid

    zero16 = jnp.zeros((16,), jnp.int32)
    ones16 = jnp.ones((16,), jnp.int32)

    def _zb(i, carry):
        hist_v[pl.ds(pl.multiple_of(i * 16, 16), 16)] = zero16
        return carry

    lax.fori_loop(0, NBINS // 16, _zb, 0)

    def _chunk(t, carry):
        chunk = t * NW + wid

        @pl.when(chunk < NCHUNK)
        def _():
            base = pl.multiple_of(chunk * K, K)
            pltpu.sync_copy(dst_hbm.at[pl.ds(base, K)], idx_v)

            def _sub(j, c2):
                idx16 = idx_v[pl.ds(pl.multiple_of(j * 16, 16), 16)]
                plsc.addupdate_scatter(hist_v, [idx16], ones16)
                return c2

            lax.fori_loop(0, K // 16, _sub, 0)

        return carry

    lax.fori_loop(0, CHUNKS_PER_W, _chunk, 0)

    pltpu.sync_copy(hist_v, stage_sh.at[sid])
    plsc.subcore_barrier()
    pltpu.sync_copy(
        stage_sh.at[:, pl.ds(sid * BPT, BPT)], red_v
    )

    def _red(i, carry):
        off = pl.ds(pl.multiple_of(i * 16, 16), 16)
        acc = zero16

        def _rows(r, a):
            return a + red_v[r, off]

        acc = lax.fori_loop(0, NS, _rows, acc)
        res_v[off] = acc.astype(jnp.float32)
        return carry

    lax.fori_loop(0, BPT // 16, _red, 0)
    pltpu.sync_copy(res_v, out_hbm.at[cid, pl.ds(sid * BPT, BPT)])


@functools.partial(
    pl.kernel,
    out_type=jax.ShapeDtypeStruct((NC, NPAD, D), jnp.float32),
    mesh=_mesh,
    scratch_types=[
        pltpu.VMEM((K,), jnp.int32),
        pltpu.VMEM((K,), jnp.int32),
        pltpu.VMEM((K, D), jnp.float32),
        pltpu.VMEM_SHARED((NPAD, D), jnp.float32),
        pltpu.SemaphoreType.DMA,
    ],
    compiler_params=pltpu.CompilerParams(needs_layout_passes=False),
)
def _scatter_call(g_hbm, src_hbm, dst_hbm, out_hbm, srcv, dstv, rows, acc, sem):
    cid = lax.axis_index("c")
    sid = lax.axis_index("s")
    wid = sid * NC + cid

    zero16 = jnp.zeros((16,), jnp.float32)

    def _zb(i, carry):
        rows[i >> 3, pl.ds(pl.multiple_of((i & 7) * 16, 16), 16)] = zero16
        return carry

    lax.fori_loop(0, K * (D // 16), _zb, 0)

    def _za(k, carry):
        pltpu.sync_copy(rows, acc.at[pl.ds(sid * RPW + k * K, K)])
        return carry

    lax.fori_loop(0, RPW // K, _za, 0)
    plsc.subcore_barrier()

    def _chunk(t, carry):
        chunk = t * NW + wid

        @pl.when(chunk < NCHUNK)
        def _():
            base = pl.multiple_of(chunk * K, K)
            pltpu.sync_copy(src_hbm.at[pl.ds(base, K)], srcv)
            pltpu.sync_copy(dst_hbm.at[pl.ds(base, K)], dstv)
            pltpu.async_copy(g_hbm.at[srcv], rows, sem).wait()
            pltpu.sync_copy(rows, acc.at[dstv], add=True)

        return carry

    lax.fori_loop(0, CHUNKS_PER_W, _chunk, 0)
    plsc.subcore_barrier()

    pltpu.sync_copy(
        acc.at[pl.ds(sid * RPW, RPW)], out_hbm.at[cid, pl.ds(sid * RPW, RPW)]
    )


_ROWS_BLK = 2000


def _mm_body(x_ref, w_ref, deg_ref, g_ref):
    dinv = lax.rsqrt(deg_ref[...])
    g_ref[...] = (
        jnp.dot(x_ref[...], w_ref[...], preferred_element_type=jnp.float32)
        * dinv
    )


def _fin_body(s0_ref, s1_ref, g_ref, deg_ref, b_ref, o_ref):
    dinv = lax.rsqrt(deg_ref[...])
    z = (s0_ref[...] + s1_ref[...] + g_ref[...]) * dinv + b_ref[...]
    o_ref[...] = jnp.where(z > 0, z, jnp.exp(jnp.minimum(z, 0.0)) - 1.0)


def kernel(x, edge_index, W, b):
    src = edge_index[0]
    dst = edge_index[1]

    hp = _deg_call(dst)
    deg = (hp[0] + hp[1]).reshape(-1)[:N] + 1.0
    deg_b = jnp.broadcast_to(deg[:, None], (N, D))

    g = pl.pallas_call(
        _mm_body,
        grid=(N // _ROWS_BLK,),
        in_specs=[
            pl.BlockSpec((_ROWS_BLK, D), lambda i: (i, 0)),
            pl.BlockSpec((D, D), lambda i: (0, 0)),
            pl.BlockSpec((_ROWS_BLK, D), lambda i: (i, 0)),
        ],
        out_specs=pl.BlockSpec((_ROWS_BLK, D), lambda i: (i, 0)),
        out_shape=jax.ShapeDtypeStruct((N, D), jnp.float32),
    )(x, W, deg_b)

    s = _scatter_call(g, src, dst)
    s0 = s[0, :N]
    s1 = s[1, :N]

    out = pl.pallas_call(
        _fin_body,
        grid=(N // _ROWS_BLK,),
        in_specs=[
            pl.BlockSpec((_ROWS_BLK, D), lambda i: (i, 0)),
            pl.BlockSpec((_ROWS_BLK, D), lambda i: (i, 0)),
            pl.BlockSpec((_ROWS_BLK, D), lambda i: (i, 0)),
            pl.BlockSpec((_ROWS_BLK, D), lambda i: (i, 0)),
            pl.BlockSpec((1, D), lambda i: (0, 0)),
        ],
        out_specs=pl.BlockSpec((_ROWS_BLK, D), lambda i: (i, 0)),
        out_shape=jax.ShapeDtypeStruct((N, D), jnp.float32),
    )(s0, s1, g, deg_b, b.reshape(1, D))

    return out
